# baseline (speedup 1.0000x reference)
import functools

import jax
import jax.numpy as jnp
from jax import lax
from jax.experimental import pallas as pl
from jax.experimental.pallas import tpu as pltpu
from jax.experimental.pallas import tpu_sc as plsc

N = 10000
NP = 10240
D = 128
E = 320000
NG = 16

NC = 2
NS = 16
W = 64
NW = 160
CH = 32
NCH = NW // CH
EPT = NW * W
ECAP = NC * NS * EPT
ROWS_PT = NP // NS

R = 1024
GRID = NP // R


def _make_sc_deg():
    mesh = plsc.VectorSubcoreMesh(core_axis_name="c", subcore_axis_name="s")
    ept = E // (NC * NS)

    @functools.partial(
        pl.kernel,
        out_type=jax.ShapeDtypeStruct((NC * NS, NP), jnp.float32),
        mesh=mesh,
        scratch_types=[
            pltpu.VMEM((ept,), jnp.int32),
            pltpu.VMEM((NP,), jnp.float32),
        ],
        compiler_params=pltpu.CompilerParams(needs_layout_passes=False),
    )
    def sc_deg(dst_hbm, zflat_hbm, out_hbm, dstv, deg):
        c = lax.axis_index("c")
        s = lax.axis_index("s")
        pltpu.sync_copy(dst_hbm.at[c, s], dstv)
        pltpu.sync_copy(zflat_hbm, deg)
        ones = jnp.ones((16,), jnp.float32)

        def body(i, carry):
            for u in range(4):
                idx = dstv[pl.ds(i * 64 + u * 16, 16)]
                plsc.addupdate_scatter(deg, [idx], ones)
            return carry

        lax.fori_loop(0, ept // 64, body, 0)
        pltpu.sync_copy(deg, out_hbm.at[c * NS + s])

    return sc_deg


_make_sc_deg = functools.cache(_make_sc_deg)


def _make_sc_scatter():
    mesh = plsc.VectorSubcoreMesh(core_axis_name="c", subcore_axis_name="s")

    @functools.partial(
        pl.kernel,
        out_type=jax.ShapeDtypeStruct((NC, NP, D), jnp.float32),
        mesh=mesh,
        scratch_types=[
            pltpu.VMEM((CH, W), jnp.int32),
            pltpu.VMEM((CH, W), jnp.int32),
            [pltpu.VMEM((W, D), jnp.float32)] * 4,
            [pltpu.SemaphoreType.DMA] * 4,
            [pltpu.SemaphoreType.DMA] * 4,
            pltpu.VMEM_SHARED((NP, D), jnp.float32),
        ],
    )
    def sc_scat(g_hbm, src_hbm, dst_hbm, z_hbm, out_hbm,
                src_idx, dst_idx, bufs, gsems, ssems, acc):
        c = lax.axis_index("c")
        s = lax.axis_index("s")
        pltpu.sync_copy(z_hbm.at[pl.ds(s * ROWS_PT, ROWS_PT)],
                        acc.at[pl.ds(s * ROWS_PT, ROWS_PT)])
        plsc.subcore_barrier()

        def g_start(w, b):
            pltpu.async_copy(g_hbm.at[src_idx.at[w]], bufs[b], gsems[b])

        def g_wait(w, b):
            pltpu.make_async_copy(g_hbm.at[src_idx.at[w]], bufs[b],
                                  gsems[b]).wait()

        def s_start(w, b):
            pltpu.async_copy(bufs[b], acc.at[dst_idx.at[w]], ssems[b],
                             add=True)

        def s_wait(w, b):
            pltpu.make_async_copy(bufs[b], acc.at[dst_idx.at[w]],
                                  ssems[b]).wait()

        def chunk(k, carry):
            pltpu.sync_copy(src_hbm.at[c, s, pl.ds(k * CH, CH)], src_idx)
            pltpu.sync_copy(dst_hbm.at[c, s, pl.ds(k * CH, CH)], dst_idx)
            g_start(0, 0)
            g_start(1, 1)

            def body(i4, carry2):
                for j in range(4):
                    w = i4 * 4 + j

                    @pl.when(w >= 2)
                    def _():
                        s_wait(w - 2, (j + 2) % 4)

                    @pl.when(w + 2 < CH)
                    def _():
                        g_start(w + 2, (j + 2) % 4)

                    g_wait(w, j)
                    s_start(w, j)
                return carry2

            lax.fori_loop(0, CH // 4, body, 0)
            s_wait(CH - 2, (CH - 2) % 4)
            s_wait(CH - 1, (CH - 1) % 4)
            return carry

        lax.fori_loop(0, NCH, chunk, 0)
        plsc.subcore_barrier()
        pltpu.sync_copy(acc.at[pl.ds(s * ROWS_PT, ROWS_PT)],
                        out_hbm.at[c, pl.ds(s * ROWS_PT, ROWS_PT)])

    return sc_scat


_make_sc_scatter = functools.cache(_make_sc_scatter)


def _tc_mm_body(x_ref, w_ref, out_ref):
    out_ref[...] = jnp.dot(x_ref[...], w_ref[...],
                           preferred_element_type=jnp.float32)


def _tc_mm(x_pad, W1):
    return pl.pallas_call(
        _tc_mm_body,
        grid=(GRID,),
        in_specs=[
            pl.BlockSpec((R, D), lambda j: (j, 0)),
            pl.BlockSpec((D, D), lambda j: (0, 0)),
        ],
        out_specs=pl.BlockSpec((R, D), lambda j: (j, 0)),
        out_shape=jax.ShapeDtypeStruct((NP, D), jnp.float32),
    )(x_pad, W1)


def _tc_pre_body(hw_ref, dp_ref, g_ref, dinv_ref):
    ones = jnp.ones((NC * NS, 1), jnp.float32)
    s_col = lax.dot_general(dp_ref[...], ones, (((0,), (0,)), ((), ())),
                            preferred_element_type=jnp.float32)
    dinv = lax.rsqrt(1.0 + s_col)
    g_ref[...] = hw_ref[...] * dinv
    dinv_ref[...] = jnp.broadcast_to(dinv, (R, D))


def _tc_pre(hw1, deg_parts):
    return pl.pallas_call(
        _tc_pre_body,
        grid=(GRID,),
        in_specs=[
            pl.BlockSpec((R, D), lambda j: (j, 0)),
            pl.BlockSpec((NC * NS, R), lambda j: (0, j)),
        ],
        out_specs=[
            pl.BlockSpec((R, D), lambda j: (j, 0)),
            pl.BlockSpec((R, D), lambda j: (j, 0)),
        ],
        out_shape=[
            jax.ShapeDtypeStruct((NP, D), jnp.float32),
            jax.ShapeDtypeStruct((NP, D), jnp.float32),
        ],
    )(hw1, deg_parts)


def _tc_b_body(acc_ref, g_ref, dinv_ref, b_ref, w_ref, out_ref):
    j = pl.program_id(0)
    a = acc_ref[0] + acc_ref[1]
    h1 = jnp.maximum(dinv_ref[...] * (a + g_ref[...]) + b_ref[...], 0.0)
    hw2 = jnp.dot(h1, w_ref[...], preferred_element_type=jnp.float32)
    g2 = dinv_ref[...] * hw2
    rows = lax.broadcasted_iota(jnp.int32, (R, D), 0) + j * R
    out_ref[...] = jnp.where(rows < N, g2, 0.0)


def _tc_b(acc1, g1, dinv_b, b1, W2):
    return pl.pallas_call(
        _tc_b_body,
        grid=(GRID,),
        in_specs=[
            pl.BlockSpec((NC, R, D), lambda j: (0, j, 0)),
            pl.BlockSpec((R, D), lambda j: (j, 0)),
            pl.BlockSpec((R, D), lambda j: (j, 0)),
            pl.BlockSpec((1, D), lambda j: (0, 0)),
            pl.BlockSpec((D, D), lambda j: (0, 0)),
        ],
        out_specs=pl.BlockSpec((R, D), lambda j: (j, 0)),
        out_shape=jax.ShapeDtypeStruct((NP, D), jnp.float32),
    )(acc1, g1, dinv_b, b1, W2)


def _tc_c_body(acc_ref, g_ref, dinv_ref, b_ref, batch_ref, out_ref):
    j = pl.program_id(0)

    @pl.when(j == 0)
    def _():
        out_ref[...] = jnp.full((NG, D), -jnp.inf, jnp.float32)

    a = acc_ref[0] + acc_ref[1]
    h2 = dinv_ref[...] * (a + g_ref[...]) + b_ref[...]
    bc = batch_ref[...]
    parts = []
    for gidx in range(NG):
        hm = jnp.where(bc == gidx, h2, -jnp.inf)
        parts.append(jnp.max(hm, axis=0, keepdims=True))
    blockmax = jnp.concatenate(parts, axis=0)
    out_ref[...] = jnp.maximum(out_ref[...], blockmax)


def _tc_c(acc2, g2, dinv_b, b2, batch_col):
    return pl.pallas_call(
        _tc_c_body,
        grid=(GRID,),
        in_specs=[
            pl.BlockSpec((NC, R, D), lambda j: (0, j, 0)),
            pl.BlockSpec((R, D), lambda j: (j, 0)),
            pl.BlockSpec((R, D), lambda j: (j, 0)),
            pl.BlockSpec((1, D), lambda j: (0, 0)),
            pl.BlockSpec((R, 1), lambda j: (j, 0)),
        ],
        out_specs=pl.BlockSpec((NG, D), lambda j: (0, 0)),
        out_shape=jax.ShapeDtypeStruct((NG, D), jnp.float32),
    )(acc2, g2, dinv_b, b2, batch_col)


def kernel(x, edge_index, batch, W1, b1, W2, b2):
    src = edge_index[0]
    dst = edge_index[1]

    pad = ECAP - E
    pad_idx = (N + (jnp.arange(pad, dtype=jnp.int32) % (NP - N)))
    srcp = jnp.concatenate([src, pad_idx]).reshape(NC, NS, NW, W)
    dstp = jnp.concatenate([dst, pad_idx]).reshape(NC, NS, NW, W)
    dst_deg = dst.reshape(NC, NS, E // (NC * NS))

    x_pad = jnp.pad(x, ((0, NP - N), (0, 0)))
    batch_col = jnp.concatenate(
        [batch, jnp.full((NP - N,), NG, jnp.int32)]).reshape(NP, 1)
    zrows = jnp.zeros((NP, D), jnp.float32)
    zflat = jnp.zeros((NP,), jnp.float32)
    b1r = b1.reshape(1, D)
    b2r = b2.reshape(1, D)

    sc_deg = _make_sc_deg()
    sc_scat = _make_sc_scatter()
    hw1 = _tc_mm(x_pad, W1)
    deg_parts = sc_deg(dst_deg, zflat)
    g1, dinv_b = _tc_pre(hw1, deg_parts)
    acc1 = sc_scat(g1, srcp, dstp, zrows)
    g2 = _tc_b(acc1, g1, dinv_b, b1r, W2)
    acc2 = sc_scat(g2, srcp, dstp, zrows)
    return _tc_c(acc2, g2, dinv_b, b2r, batch_col)


# speedup vs baseline: 1.0005x; 1.0005x over previous
import functools

import jax
import jax.numpy as jnp
from jax import lax
from jax.experimental import pallas as pl
from jax.experimental.pallas import tpu as pltpu
from jax.experimental.pallas import tpu_sc as plsc

N = 10000
NP = 10240
D = 128
E = 320000
NG = 16

NC = 2
NS = 16
W = 64
NW = 160
CH = 32
NCH = NW // CH
EPT = NW * W
ECAP = NC * NS * EPT
ROWS_PT = NP // NS

R = 1024
GRID = NP // R


def _make_sc_deg():
    mesh = plsc.VectorSubcoreMesh(core_axis_name="c", subcore_axis_name="s")
    ept = E // (NC * NS)

    @functools.partial(
        pl.kernel,
        out_type=jax.ShapeDtypeStruct((NC * NS, NP), jnp.float32),
        mesh=mesh,
        scratch_types=[
            pltpu.VMEM((ept,), jnp.int32),
            pltpu.VMEM((NP,), jnp.float32),
        ],
        compiler_params=pltpu.CompilerParams(needs_layout_passes=False),
    )
    def sc_deg(dst_hbm, zflat_hbm, out_hbm, dstv, deg):
        c = lax.axis_index("c")
        s = lax.axis_index("s")
        pltpu.sync_copy(dst_hbm.at[c, s], dstv)
        pltpu.sync_copy(zflat_hbm, deg)
        ones = jnp.ones((16,), jnp.float32)

        def body(i, carry):
            for u in range(5):
                idx = dstv[pl.ds(i * 80 + u * 16, 16)]
                plsc.addupdate_scatter(deg, [idx], ones)
            return carry

        lax.fori_loop(0, ept // 80, body, 0)
        pltpu.sync_copy(deg, out_hbm.at[c * NS + s])

    return sc_deg


_make_sc_deg = functools.cache(_make_sc_deg)


def _make_sc_scatter():
    mesh = plsc.VectorSubcoreMesh(core_axis_name="c", subcore_axis_name="s")

    @functools.partial(
        pl.kernel,
        out_type=jax.ShapeDtypeStruct((NC, NP, D), jnp.float32),
        mesh=mesh,
        scratch_types=[
            pltpu.VMEM((CH, W), jnp.int32),
            pltpu.VMEM((CH, W), jnp.int32),
            [pltpu.VMEM((W, D), jnp.float32)] * 4,
            [pltpu.SemaphoreType.DMA] * 4,
            [pltpu.SemaphoreType.DMA] * 4,
            pltpu.VMEM_SHARED((NP, D), jnp.float32),
        ],
    )
    def sc_scat(g_hbm, src_hbm, dst_hbm, z_hbm, out_hbm,
                src_idx, dst_idx, bufs, gsems, ssems, acc):
        c = lax.axis_index("c")
        s = lax.axis_index("s")
        pltpu.sync_copy(z_hbm.at[pl.ds(s * ROWS_PT, ROWS_PT)],
                        acc.at[pl.ds(s * ROWS_PT, ROWS_PT)])
        plsc.subcore_barrier()

        def g_start(w, b):
            pltpu.async_copy(g_hbm.at[src_idx.at[w]], bufs[b], gsems[b])

        def g_wait(w, b):
            pltpu.make_async_copy(g_hbm.at[src_idx.at[w]], bufs[b],
                                  gsems[b]).wait()

        def s_start(w, b):
            pltpu.async_copy(bufs[b], acc.at[dst_idx.at[w]], ssems[b],
                             add=True)

        def s_wait(w, b):
            pltpu.make_async_copy(bufs[b], acc.at[dst_idx.at[w]],
                                  ssems[b]).wait()

        def chunk(k, carry):
            pltpu.sync_copy(src_hbm.at[c, s, pl.ds(k * CH, CH)], src_idx)
            pltpu.sync_copy(dst_hbm.at[c, s, pl.ds(k * CH, CH)], dst_idx)
            g_start(0, 0)
            g_start(1, 1)

            def body(i4, carry2):
                for j in range(4):
                    w = i4 * 4 + j

                    @pl.when(w >= 2)
                    def _():
                        s_wait(w - 2, (j + 2) % 4)

                    @pl.when(w + 2 < CH)
                    def _():
                        g_start(w + 2, (j + 2) % 4)

                    g_wait(w, j)
                    s_start(w, j)
                return carry2

            lax.fori_loop(0, CH // 4, body, 0)
            s_wait(CH - 2, (CH - 2) % 4)
            s_wait(CH - 1, (CH - 1) % 4)
            return carry

        lax.fori_loop(0, NCH, chunk, 0)
        plsc.subcore_barrier()
        pltpu.sync_copy(acc.at[pl.ds(s * ROWS_PT, ROWS_PT)],
                        out_hbm.at[c, pl.ds(s * ROWS_PT, ROWS_PT)])

    return sc_scat


_make_sc_scatter = functools.cache(_make_sc_scatter)


def _tc_mm_body(x_ref, w_ref, out_ref):
    out_ref[...] = jnp.dot(x_ref[...], w_ref[...],
                           preferred_element_type=jnp.float32)


def _tc_mm(x_pad, W1):
    return pl.pallas_call(
        _tc_mm_body,
        grid=(GRID,),
        in_specs=[
            pl.BlockSpec((R, D), lambda j: (j, 0)),
            pl.BlockSpec((D, D), lambda j: (0, 0)),
        ],
        out_specs=pl.BlockSpec((R, D), lambda j: (j, 0)),
        out_shape=jax.ShapeDtypeStruct((NP, D), jnp.float32),
    )(x_pad, W1)


def _tc_pre_body(hw_ref, dp_ref, g_ref, dinv_ref):
    ones = jnp.ones((NC * NS, 1), jnp.float32)
    s_col = lax.dot_general(dp_ref[...], ones, (((0,), (0,)), ((), ())),
                            preferred_element_type=jnp.float32)
    dinv = lax.rsqrt(1.0 + s_col)
    g_ref[...] = hw_ref[...] * dinv
    dinv_ref[...] = jnp.broadcast_to(dinv, (R, D))


def _tc_pre(hw1, deg_parts):
    return pl.pallas_call(
        _tc_pre_body,
        grid=(GRID,),
        in_specs=[
            pl.BlockSpec((R, D), lambda j: (j, 0)),
            pl.BlockSpec((NC * NS, R), lambda j: (0, j)),
        ],
        out_specs=[
            pl.BlockSpec((R, D), lambda j: (j, 0)),
            pl.BlockSpec((R, D), lambda j: (j, 0)),
        ],
        out_shape=[
            jax.ShapeDtypeStruct((NP, D), jnp.float32),
            jax.ShapeDtypeStruct((NP, D), jnp.float32),
        ],
    )(hw1, deg_parts)


def _tc_b_body(acc_ref, g_ref, dinv_ref, b_ref, w_ref, out_ref):
    j = pl.program_id(0)
    a = acc_ref[0] + acc_ref[1]
    h1 = jnp.maximum(dinv_ref[...] * (a + g_ref[...]) + b_ref[...], 0.0)
    hw2 = jnp.dot(h1, w_ref[...], preferred_element_type=jnp.float32)
    g2 = dinv_ref[...] * hw2
    rows = lax.broadcasted_iota(jnp.int32, (R, D), 0) + j * R
    out_ref[...] = jnp.where(rows < N, g2, 0.0)


def _tc_b(acc1, g1, dinv_b, b1, W2):
    return pl.pallas_call(
        _tc_b_body,
        grid=(GRID,),
        in_specs=[
            pl.BlockSpec((NC, R, D), lambda j: (0, j, 0)),
            pl.BlockSpec((R, D), lambda j: (j, 0)),
            pl.BlockSpec((R, D), lambda j: (j, 0)),
            pl.BlockSpec((1, D), lambda j: (0, 0)),
            pl.BlockSpec((D, D), lambda j: (0, 0)),
        ],
        out_specs=pl.BlockSpec((R, D), lambda j: (j, 0)),
        out_shape=jax.ShapeDtypeStruct((NP, D), jnp.float32),
    )(acc1, g1, dinv_b, b1, W2)


def _tc_c_body(acc_ref, g_ref, dinv_ref, b_ref, batch_ref, out_ref):
    j = pl.program_id(0)

    @pl.when(j == 0)
    def _():
        out_ref[...] = jnp.full((NG, D), -jnp.inf, jnp.float32)

    a = acc_ref[0] + acc_ref[1]
    h2 = dinv_ref[...] * (a + g_ref[...]) + b_ref[...]
    bc = batch_ref[...]
    parts = []
    for gidx in range(NG):
        hm = jnp.where(bc == gidx, h2, -jnp.inf)
        parts.append(jnp.max(hm, axis=0, keepdims=True))
    blockmax = jnp.concatenate(parts, axis=0)
    out_ref[...] = jnp.maximum(out_ref[...], blockmax)


def _tc_c(acc2, g2, dinv_b, b2, batch_col):
    return pl.pallas_call(
        _tc_c_body,
        grid=(GRID,),
        in_specs=[
            pl.BlockSpec((NC, R, D), lambda j: (0, j, 0)),
            pl.BlockSpec((R, D), lambda j: (j, 0)),
            pl.BlockSpec((R, D), lambda j: (j, 0)),
            pl.BlockSpec((1, D), lambda j: (0, 0)),
            pl.BlockSpec((R, 1), lambda j: (j, 0)),
        ],
        out_specs=pl.BlockSpec((NG, D), lambda j: (0, 0)),
        out_shape=jax.ShapeDtypeStruct((NG, D), jnp.float32),
    )(acc2, g2, dinv_b, b2, batch_col)


def kernel(x, edge_index, batch, W1, b1, W2, b2):
    src = edge_index[0]
    dst = edge_index[1]

    pad = ECAP - E
    pad_idx = (N + (jnp.arange(pad, dtype=jnp.int32) % (NP - N)))
    srcp = jnp.concatenate([src, pad_idx]).reshape(NC, NS, NW, W)
    dstp = jnp.concatenate([dst, pad_idx]).reshape(NC, NS, NW, W)
    dst_deg = dst.reshape(NC, NS, E // (NC * NS))

    x_pad = jnp.pad(x, ((0, NP - N), (0, 0)))
    batch_col = jnp.concatenate(
        [batch, jnp.full((NP - N,), NG, jnp.int32)]).reshape(NP, 1)
    zrows = jnp.zeros((NP, D), jnp.float32)
    zflat = jnp.zeros((NP,), jnp.float32)
    b1r = b1.reshape(1, D)
    b2r = b2.reshape(1, D)

    sc_deg = _make_sc_deg()
    sc_scat = _make_sc_scatter()
    hw1 = _tc_mm(x_pad, W1)
    deg_parts = sc_deg(dst_deg, zflat)
    g1, dinv_b = _tc_pre(hw1, deg_parts)
    acc1 = sc_scat(g1, srcp, dstp, zrows)
    g2 = _tc_b(acc1, g1, dinv_b, b1r, W2)
    acc2 = sc_scat(g2, srcp, dstp, zrows)
    return _tc_c(acc2, g2, dinv_b, b2r, batch_col)


# speedup vs baseline: 1.0470x; 1.0465x over previous
import functools

import jax
import jax.numpy as jnp
from jax import lax
from jax.experimental import pallas as pl
from jax.experimental.pallas import tpu as pltpu
from jax.experimental.pallas import tpu_sc as plsc

N = 10000
NP = 10240
D = 128
E = 320000
NG = 16

NC = 2
NS = 16
W = 64
NW = 160
CH = 32
NCH = NW // CH
EPT = NW * W
ECAP = NC * NS * EPT
ROWS_PT = NP // NS

R = 1024
GRID = NP // R


def _make_sc_deg():
    mesh = plsc.VectorSubcoreMesh(core_axis_name="c", subcore_axis_name="s")
    ept = E // (NC * NS)

    @functools.partial(
        pl.kernel,
        out_type=jax.ShapeDtypeStruct((NC * NS, NP), jnp.float32),
        mesh=mesh,
        scratch_types=[
            pltpu.VMEM((ept,), jnp.int32),
            pltpu.VMEM((NP,), jnp.float32),
        ],
        compiler_params=pltpu.CompilerParams(needs_layout_passes=False),
    )
    def sc_deg(dst_hbm, zflat_hbm, out_hbm, dstv, deg):
        c = lax.axis_index("c")
        s = lax.axis_index("s")
        pltpu.sync_copy(dst_hbm.at[c, s], dstv)
        pltpu.sync_copy(zflat_hbm, deg)
        ones = jnp.ones((16,), jnp.float32)

        def body(i, carry):
            idx = dstv[pl.ds(i * 16, 16)]
            plsc.addupdate_scatter(deg, [idx], ones)
            return carry

        lax.fori_loop(0, ept // 16, body, 0)
        pltpu.sync_copy(deg, out_hbm.at[c * NS + s])

    return sc_deg


_make_sc_deg = functools.cache(_make_sc_deg)


def _make_sc_scatter():
    mesh = plsc.VectorSubcoreMesh(core_axis_name="c", subcore_axis_name="s")

    @functools.partial(
        pl.kernel,
        out_type=jax.ShapeDtypeStruct((NC, NP, D), jnp.float32),
        mesh=mesh,
        scratch_types=[
            pltpu.VMEM((CH, W), jnp.int32),
            pltpu.VMEM((CH, W), jnp.int32),
            [pltpu.VMEM((W, D), jnp.float32)] * 4,
            [pltpu.SemaphoreType.DMA] * 4,
            [pltpu.SemaphoreType.DMA] * 4,
            pltpu.VMEM_SHARED((NP, D), jnp.float32),
        ],
    )
    def sc_scat(g_hbm, src_hbm, dst_hbm, z_hbm, out_hbm,
                src_idx, dst_idx, bufs, gsems, ssems, acc):
        c = lax.axis_index("c")
        s = lax.axis_index("s")
        pltpu.sync_copy(z_hbm.at[pl.ds(s * ROWS_PT, ROWS_PT)],
                        acc.at[pl.ds(s * ROWS_PT, ROWS_PT)])
        plsc.subcore_barrier()

        def g_start(w, b):
            pltpu.async_copy(g_hbm.at[src_idx.at[w]], bufs[b], gsems[b])

        def g_wait(w, b):
            pltpu.make_async_copy(g_hbm.at[src_idx.at[w]], bufs[b],
                                  gsems[b]).wait()

        def s_start(w, b):
            pltpu.async_copy(bufs[b], acc.at[dst_idx.at[w]], ssems[b],
                             add=True)

        def s_wait(w, b):
            pltpu.make_async_copy(bufs[b], acc.at[dst_idx.at[w]],
                                  ssems[b]).wait()

        def chunk(k, carry):
            pltpu.sync_copy(src_hbm.at[c, s, pl.ds(k * CH, CH)], src_idx)
            pltpu.sync_copy(dst_hbm.at[c, s, pl.ds(k * CH, CH)], dst_idx)
            g_start(0, 0)
            g_start(1, 1)

            def body(i4, carry2):
                for j in range(4):
                    w = i4 * 4 + j

                    @pl.when(w >= 2)
                    def _():
                        s_wait(w - 2, (j + 2) % 4)

                    @pl.when(w + 2 < CH)
                    def _():
                        g_start(w + 2, (j + 2) % 4)

                    g_wait(w, j)
                    s_start(w, j)
                return carry2

            lax.fori_loop(0, CH // 4, body, 0)
            s_wait(CH - 2, (CH - 2) % 4)
            s_wait(CH - 1, (CH - 1) % 4)
            return carry

        lax.fori_loop(0, NCH, chunk, 0)
        plsc.subcore_barrier()
        pltpu.sync_copy(acc.at[pl.ds(s * ROWS_PT, ROWS_PT)],
                        out_hbm.at[c, pl.ds(s * ROWS_PT, ROWS_PT)])

    return sc_scat


_make_sc_scatter = functools.cache(_make_sc_scatter)


def _tc_a_body(x_ref, w_ref, dp_ref, g_ref, dinv_ref):
    ones = jnp.ones((NC * NS, 1), jnp.float32)
    s_col = lax.dot_general(dp_ref[...], ones, (((0,), (0,)), ((), ())),
                            preferred_element_type=jnp.float32)
    dinv = lax.rsqrt(1.0 + s_col)
    hw = jnp.dot(x_ref[...], w_ref[...], preferred_element_type=jnp.float32)
    g_ref[...] = hw * dinv
    dinv_ref[...] = jnp.broadcast_to(dinv, (R, D))


def _tc_a(x_pad, W1, deg_parts):
    return pl.pallas_call(
        _tc_a_body,
        grid=(GRID,),
        in_specs=[
            pl.BlockSpec((R, D), lambda j: (j, 0)),
            pl.BlockSpec((D, D), lambda j: (0, 0)),
            pl.BlockSpec((NC * NS, R), lambda j: (0, j)),
        ],
        out_specs=[
            pl.BlockSpec((R, D), lambda j: (j, 0)),
            pl.BlockSpec((R, D), lambda j: (j, 0)),
        ],
        out_shape=[
            jax.ShapeDtypeStruct((NP, D), jnp.float32),
            jax.ShapeDtypeStruct((NP, D), jnp.float32),
        ],
    )(x_pad, W1, deg_parts)


def _tc_b_body(acc_ref, g_ref, dinv_ref, b_ref, w_ref, out_ref):
    j = pl.program_id(0)
    a = acc_ref[0] + acc_ref[1]
    h1 = jnp.maximum(dinv_ref[...] * (a + g_ref[...]) + b_ref[...], 0.0)
    hw2 = jnp.dot(h1, w_ref[...], preferred_element_type=jnp.float32)
    g2 = dinv_ref[...] * hw2
    rows = lax.broadcasted_iota(jnp.int32, (R, D), 0) + j * R
    out_ref[...] = jnp.where(rows < N, g2, 0.0)


def _tc_b(acc1, g1, dinv_b, b1, W2):
    return pl.pallas_call(
        _tc_b_body,
        grid=(GRID,),
        in_specs=[
            pl.BlockSpec((NC, R, D), lambda j: (0, j, 0)),
            pl.BlockSpec((R, D), lambda j: (j, 0)),
            pl.BlockSpec((R, D), lambda j: (j, 0)),
            pl.BlockSpec((1, D), lambda j: (0, 0)),
            pl.BlockSpec((D, D), lambda j: (0, 0)),
        ],
        out_specs=pl.BlockSpec((R, D), lambda j: (j, 0)),
        out_shape=jax.ShapeDtypeStruct((NP, D), jnp.float32),
    )(acc1, g1, dinv_b, b1, W2)


def _tc_c_body(acc_ref, g_ref, dinv_ref, b_ref, batch_ref, out_ref):
    j = pl.program_id(0)

    @pl.when(j == 0)
    def _():
        out_ref[...] = jnp.full((NG, D), -jnp.inf, jnp.float32)

    a = acc_ref[0] + acc_ref[1]
    h2 = dinv_ref[...] * (a + g_ref[...]) + b_ref[...]
    bc = batch_ref[...]
    CHK = 64
    for cidx in range(R // CHK):
        rows = h2[cidx * CHK:(cidx + 1) * CHK]
        bchunk = bc[cidx * CHK:(cidx + 1) * CHK]
        g0 = batch_ref[cidx * CHK, 0]
        g1 = batch_ref[cidx * CHK + CHK - 1, 0]

        @pl.when(jnp.logical_and(g0 == g1, g0 < NG))
        def _():
            cmax = jnp.max(rows, axis=0, keepdims=True)
            cur = out_ref[pl.ds(g0, 1), :]
            out_ref[pl.ds(g0, 1), :] = jnp.maximum(cur, cmax)

        @pl.when(g0 != g1)
        def _():
            parts = []
            for gidx in range(NG):
                hm = jnp.where(bchunk == gidx, rows, -jnp.inf)
                parts.append(jnp.max(hm, axis=0, keepdims=True))
            blockmax = jnp.concatenate(parts, axis=0)
            out_ref[...] = jnp.maximum(out_ref[...], blockmax)


def _tc_c(acc2, g2, dinv_b, b2, batch_col):
    return pl.pallas_call(
        _tc_c_body,
        grid=(GRID,),
        in_specs=[
            pl.BlockSpec((NC, R, D), lambda j: (0, j, 0)),
            pl.BlockSpec((R, D), lambda j: (j, 0)),
            pl.BlockSpec((R, D), lambda j: (j, 0)),
            pl.BlockSpec((1, D), lambda j: (0, 0)),
            pl.BlockSpec((R, 1), lambda j: (j, 0)),
        ],
        out_specs=pl.BlockSpec((NG, D), lambda j: (0, 0)),
        out_shape=jax.ShapeDtypeStruct((NG, D), jnp.float32),
    )(acc2, g2, dinv_b, b2, batch_col)


def kernel(x, edge_index, batch, W1, b1, W2, b2):
    src = edge_index[0]
    dst = edge_index[1]

    pad = ECAP - E
    pad_idx = (N + (jnp.arange(pad, dtype=jnp.int32) % (NP - N)))
    srcp = jnp.concatenate([src, pad_idx]).reshape(NC, NS, NW, W)
    dstp = jnp.concatenate([dst, pad_idx]).reshape(NC, NS, NW, W)
    dst_deg = dst.reshape(NC, NS, E // (NC * NS))

    x_pad = jnp.pad(x, ((0, NP - N), (0, 0)))
    batch_col = jnp.concatenate(
        [batch, jnp.full((NP - N,), NG, jnp.int32)]).reshape(NP, 1)
    zrows = jnp.zeros((NP, D), jnp.float32)
    zflat = jnp.zeros((NP,), jnp.float32)
    b1r = b1.reshape(1, D)
    b2r = b2.reshape(1, D)

    sc_deg = _make_sc_deg()
    sc_scat = _make_sc_scatter()
    deg_parts = sc_deg(dst_deg, zflat)
    g1, dinv_b = _tc_a(x_pad, W1, deg_parts)
    acc1 = sc_scat(g1, srcp, dstp, zrows)
    g2 = _tc_b(acc1, g1, dinv_b, b1r, W2)
    acc2 = sc_scat(g2, srcp, dstp, zrows)
    return _tc_c(acc2, g2, dinv_b, b2r, batch_col)


# speedup vs baseline: 1.0489x; 1.0019x over previous
import functools

import jax
import jax.numpy as jnp
from jax import lax
from jax.experimental import pallas as pl
from jax.experimental.pallas import tpu as pltpu
from jax.experimental.pallas import tpu_sc as plsc

N = 10000
NP = 10240
D = 128
E = 320000
NG = 16

NC = 2
NS = 16
W = 64
NW = 160
CH = 32
NCH = NW // CH
EPT = NW * W
ECAP = NC * NS * EPT
ROWS_PT = NP // NS

R = 1024
GRID = NP // R


def _make_sc_deg():
    mesh = plsc.VectorSubcoreMesh(core_axis_name="c", subcore_axis_name="s")
    ept = E // (NC * NS)

    @functools.partial(
        pl.kernel,
        out_type=jax.ShapeDtypeStruct((NC * NS, NP), jnp.float32),
        mesh=mesh,
        scratch_types=[
            pltpu.VMEM((ept,), jnp.int32),
            pltpu.VMEM((NP,), jnp.float32),
        ],
        compiler_params=pltpu.CompilerParams(needs_layout_passes=False),
    )
    def sc_deg(dst_hbm, zflat_hbm, out_hbm, dstv, deg):
        c = lax.axis_index("c")
        s = lax.axis_index("s")
        pltpu.sync_copy(dst_hbm.at[c, s], dstv)
        pltpu.sync_copy(zflat_hbm, deg)
        ones = jnp.ones((16,), jnp.float32)

        def body(i, carry):
            for u in range(5):
                idx = dstv[pl.ds(i * 80 + u * 16, 16)]
                plsc.addupdate_scatter(deg, [idx], ones)
            return carry

        lax.fori_loop(0, ept // 80, body, 0)
        pltpu.sync_copy(deg, out_hbm.at[c * NS + s])

    return sc_deg


_make_sc_deg = functools.cache(_make_sc_deg)


def _make_sc_scatter():
    mesh = plsc.VectorSubcoreMesh(core_axis_name="c", subcore_axis_name="s")

    @functools.partial(
        pl.kernel,
        out_type=jax.ShapeDtypeStruct((NC, NP, D), jnp.float32),
        mesh=mesh,
        scratch_types=[
            pltpu.VMEM((CH, W), jnp.int32),
            pltpu.VMEM((CH, W), jnp.int32),
            [pltpu.VMEM((W, D), jnp.float32)] * 4,
            [pltpu.SemaphoreType.DMA] * 4,
            [pltpu.SemaphoreType.DMA] * 4,
            pltpu.VMEM_SHARED((NP, D), jnp.float32),
        ],
    )
    def sc_scat(g_hbm, src_hbm, dst_hbm, z_hbm, out_hbm,
                src_idx, dst_idx, bufs, gsems, ssems, acc):
        c = lax.axis_index("c")
        s = lax.axis_index("s")
        pltpu.sync_copy(z_hbm.at[pl.ds(s * ROWS_PT, ROWS_PT)],
                        acc.at[pl.ds(s * ROWS_PT, ROWS_PT)])
        plsc.subcore_barrier()

        def g_start(w, b):
            pltpu.async_copy(g_hbm.at[src_idx.at[w]], bufs[b], gsems[b])

        def g_wait(w, b):
            pltpu.make_async_copy(g_hbm.at[src_idx.at[w]], bufs[b],
                                  gsems[b]).wait()

        def s_start(w, b):
            pltpu.async_copy(bufs[b], acc.at[dst_idx.at[w]], ssems[b],
                             add=True)

        def s_wait(w, b):
            pltpu.make_async_copy(bufs[b], acc.at[dst_idx.at[w]],
                                  ssems[b]).wait()

        def chunk(k, carry):
            pltpu.sync_copy(src_hbm.at[c, s, pl.ds(k * CH, CH)], src_idx)
            pltpu.sync_copy(dst_hbm.at[c, s, pl.ds(k * CH, CH)], dst_idx)
            g_start(0, 0)
            g_start(1, 1)

            def body(i4, carry2):
                for j in range(4):
                    w = i4 * 4 + j

                    @pl.when(w >= 2)
                    def _():
                        s_wait(w - 2, (j + 2) % 4)

                    @pl.when(w + 2 < CH)
                    def _():
                        g_start(w + 2, (j + 2) % 4)

                    g_wait(w, j)
                    s_start(w, j)
                return carry2

            lax.fori_loop(0, CH // 4, body, 0)
            s_wait(CH - 2, (CH - 2) % 4)
            s_wait(CH - 1, (CH - 1) % 4)
            return carry

        lax.fori_loop(0, NCH, chunk, 0)
        plsc.subcore_barrier()
        pltpu.sync_copy(acc.at[pl.ds(s * ROWS_PT, ROWS_PT)],
                        out_hbm.at[c, pl.ds(s * ROWS_PT, ROWS_PT)])

    return sc_scat


_make_sc_scatter = functools.cache(_make_sc_scatter)


def _tc_a_body(x_ref, w_ref, dp_ref, g_ref, dinv_ref):
    ones = jnp.ones((NC * NS, 1), jnp.float32)
    s_col = lax.dot_general(dp_ref[...], ones, (((0,), (0,)), ((), ())),
                            preferred_element_type=jnp.float32)
    dinv = lax.rsqrt(1.0 + s_col)
    hw = jnp.dot(x_ref[...], w_ref[...], preferred_element_type=jnp.float32)
    g_ref[...] = hw * dinv
    dinv_ref[...] = jnp.broadcast_to(dinv, (R, D))


def _tc_a(x_pad, W1, deg_parts):
    return pl.pallas_call(
        _tc_a_body,
        grid=(GRID,),
        in_specs=[
            pl.BlockSpec((R, D), lambda j: (j, 0)),
            pl.BlockSpec((D, D), lambda j: (0, 0)),
            pl.BlockSpec((NC * NS, R), lambda j: (0, j)),
        ],
        out_specs=[
            pl.BlockSpec((R, D), lambda j: (j, 0)),
            pl.BlockSpec((R, D), lambda j: (j, 0)),
        ],
        out_shape=[
            jax.ShapeDtypeStruct((NP, D), jnp.float32),
            jax.ShapeDtypeStruct((NP, D), jnp.float32),
        ],
    )(x_pad, W1, deg_parts)


def _tc_b_body(acc_ref, g_ref, dinv_ref, b_ref, w_ref, out_ref):
    j = pl.program_id(0)
    a = acc_ref[0] + acc_ref[1]
    h1 = jnp.maximum(dinv_ref[...] * (a + g_ref[...]) + b_ref[...], 0.0)
    hw2 = jnp.dot(h1, w_ref[...], preferred_element_type=jnp.float32)
    g2 = dinv_ref[...] * hw2
    rows = lax.broadcasted_iota(jnp.int32, (R, D), 0) + j * R
    out_ref[...] = jnp.where(rows < N, g2, 0.0)


def _tc_b(acc1, g1, dinv_b, b1, W2):
    return pl.pallas_call(
        _tc_b_body,
        grid=(GRID,),
        in_specs=[
            pl.BlockSpec((NC, R, D), lambda j: (0, j, 0)),
            pl.BlockSpec((R, D), lambda j: (j, 0)),
            pl.BlockSpec((R, D), lambda j: (j, 0)),
            pl.BlockSpec((1, D), lambda j: (0, 0)),
            pl.BlockSpec((D, D), lambda j: (0, 0)),
        ],
        out_specs=pl.BlockSpec((R, D), lambda j: (j, 0)),
        out_shape=jax.ShapeDtypeStruct((NP, D), jnp.float32),
    )(acc1, g1, dinv_b, b1, W2)


def _tc_c_body(acc_ref, g_ref, dinv_ref, b_ref, batch_ref, out_ref):
    j = pl.program_id(0)

    @pl.when(j == 0)
    def _():
        out_ref[...] = jnp.full((NG, D), -jnp.inf, jnp.float32)

    a = acc_ref[0] + acc_ref[1]
    h2 = dinv_ref[...] * (a + g_ref[...]) + b_ref[...]
    bc = batch_ref[...]
    CHK = 64
    for cidx in range(R // CHK):
        rows = h2[cidx * CHK:(cidx + 1) * CHK]
        bchunk = bc[cidx * CHK:(cidx + 1) * CHK]
        g0 = batch_ref[cidx * CHK, 0]
        g1 = batch_ref[cidx * CHK + CHK - 1, 0]

        @pl.when(jnp.logical_and(g0 == g1, g0 < NG))
        def _():
            cmax = jnp.max(rows, axis=0, keepdims=True)
            cur = out_ref[pl.ds(g0, 1), :]
            out_ref[pl.ds(g0, 1), :] = jnp.maximum(cur, cmax)

        @pl.when(g0 != g1)
        def _():
            parts = []
            for gidx in range(NG):
                hm = jnp.where(bchunk == gidx, rows, -jnp.inf)
                parts.append(jnp.max(hm, axis=0, keepdims=True))
            blockmax = jnp.concatenate(parts, axis=0)
            out_ref[...] = jnp.maximum(out_ref[...], blockmax)


def _tc_c(acc2, g2, dinv_b, b2, batch_col):
    return pl.pallas_call(
        _tc_c_body,
        grid=(GRID,),
        in_specs=[
            pl.BlockSpec((NC, R, D), lambda j: (0, j, 0)),
            pl.BlockSpec((R, D), lambda j: (j, 0)),
            pl.BlockSpec((R, D), lambda j: (j, 0)),
            pl.BlockSpec((1, D), lambda j: (0, 0)),
            pl.BlockSpec((R, 1), lambda j: (j, 0)),
        ],
        out_specs=pl.BlockSpec((NG, D), lambda j: (0, 0)),
        out_shape=jax.ShapeDtypeStruct((NG, D), jnp.float32),
    )(acc2, g2, dinv_b, b2, batch_col)


def kernel(x, edge_index, batch, W1, b1, W2, b2):
    src = edge_index[0]
    dst = edge_index[1]

    pad = ECAP - E
    pad_idx = (N + (jnp.arange(pad, dtype=jnp.int32) % (NP - N)))
    srcp = jnp.concatenate([src, pad_idx]).reshape(NC, NS, NW, W)
    dstp = jnp.concatenate([dst, pad_idx]).reshape(NC, NS, NW, W)
    dst_deg = dst.reshape(NC, NS, E // (NC * NS))

    x_pad = jnp.pad(x, ((0, NP - N), (0, 0)))
    batch_col = jnp.concatenate(
        [batch, jnp.full((NP - N,), NG, jnp.int32)]).reshape(NP, 1)
    zrows = jnp.zeros((NP, D), jnp.float32)
    zflat = jnp.zeros((NP,), jnp.float32)
    b1r = b1.reshape(1, D)
    b2r = b2.reshape(1, D)

    sc_deg = _make_sc_deg()
    sc_scat = _make_sc_scatter()
    deg_parts = sc_deg(dst_deg, zflat)
    g1, dinv_b = _tc_a(x_pad, W1, deg_parts)
    acc1 = sc_scat(g1, srcp, dstp, zrows)
    g2 = _tc_b(acc1, g1, dinv_b, b1r, W2)
    acc2 = sc_scat(g2, srcp, dstp, zrows)
    return _tc_c(acc2, g2, dinv_b, b2r, batch_col)
